# Initial kernel scaffold; baseline (speedup 1.0000x reference)
import jax
import jax.numpy as jnp
from jax.experimental import pallas as pl


def kernel(input_feature, edge_index, adj_values, W, b):
    raise NotImplementedError("write your pallas kernel here")


# speedup vs baseline: 6.5071x; 6.5071x over previous
import functools

import jax
import jax.numpy as jnp
from jax import lax
from jax.experimental import pallas as pl
from jax.experimental.pallas import tpu as pltpu
from jax.experimental.pallas import tpu_sc as plsc

NC = 2
NS = 16
NW = NC * NS
CHUNK = 128
SBLK = 16
LANES = 16

_BCAST_DNUMS = lax.GatherDimensionNumbers(
    offset_dims=(), collapsed_slice_dims=(0,), start_index_map=(0,)
)


def _bcast_lane(vec16, k):
    idx = jnp.full((LANES, 1), k, jnp.int32)
    return lax.gather(
        vec16, idx, _BCAST_DNUMS, (1,),
        mode=lax.GatherScatterMode.PROMISE_IN_BOUNDS,
    )


def _matmul(x, w):
    n, d_in = x.shape
    d_out = w.shape[1]
    bm = 400
    grid = n // bm

    def body(x_ref, w_ref, o_ref):
        o_ref[:] = jnp.dot(x_ref[:], w_ref[:], preferred_element_type=jnp.float32)

    return pl.pallas_call(
        body,
        grid=(grid,),
        in_specs=[
            pl.BlockSpec((bm, d_in), lambda i: (i, 0)),
            pl.BlockSpec((d_in, d_out), lambda i: (0, 0)),
        ],
        out_specs=pl.BlockSpec((bm, d_out), lambda i: (i, 0)),
        out_shape=jax.ShapeDtypeStruct((n, d_out), jnp.float32),
    )(x, w)


def _combine(partials, b2d):
    _, n, d = partials.shape
    bm = 2000
    grid = n // bm

    def body(p_ref, b_ref, o_ref):
        o_ref[:] = p_ref[0] + p_ref[1] + b_ref[:]

    return pl.pallas_call(
        body,
        grid=(grid,),
        in_specs=[
            pl.BlockSpec((2, bm, d), lambda i: (0, i, 0)),
            pl.BlockSpec((1, d), lambda i: (0, 0)),
        ],
        out_specs=pl.BlockSpec((bm, d), lambda i: (i, 0)),
        out_shape=jax.ShapeDtypeStruct((n, d), jnp.float32),
    )(partials, b2d)


def _make_aggregate(n, d, nch):
    mesh = plsc.VectorSubcoreMesh(core_axis_name="c", subcore_axis_name="s")
    wtiles = 10
    rows_per_tile = n // wtiles
    dslices = d // LANES
    groups = CHUNK // LANES
    nblk = nch // SBLK

    @functools.partial(
        pl.kernel,
        out_type=jax.ShapeDtypeStruct((NC, n, d), jnp.float32),
        mesh=mesh,
        scratch_types=[
            pltpu.VMEM((SBLK, CHUNK), jnp.int32),
            pltpu.VMEM((SBLK, CHUNK), jnp.int32),
            pltpu.VMEM((SBLK, CHUNK), jnp.float32),
            pltpu.VMEM((CHUNK, d), jnp.float32),
            pltpu.VMEM_SHARED((n, d), jnp.float32),
            pltpu.SemaphoreType.DMA,
        ],
    )
    def agg(support_hbm, src_hbm, dst_hbm, val_hbm, out_hbm,
            src_v, dst_v, val_v, rows_v, acc, sem):
        cid = lax.axis_index("c")
        sid = lax.axis_index("s")
        wid = sid * NC + cid

        @pl.loop(0, CHUNK)
        def _zero(r):
            for c in range(dslices):
                rows_v[r, pl.ds(c * LANES, LANES)] = jnp.zeros((LANES,), jnp.float32)

        @pl.when(sid < wtiles)
        def _zacc_all():
            @pl.loop(0, rows_per_tile // CHUNK)
            def _zacc(r):
                pltpu.sync_copy(zero_src := rows_v,
                                acc.at[pl.ds(sid * rows_per_tile + r * CHUNK, CHUNK)])
            rem = rows_per_tile % CHUNK
            if rem:
                pltpu.sync_copy(
                    rows_v.at[pl.ds(0, rem)],
                    acc.at[pl.ds(sid * rows_per_tile + (rows_per_tile // CHUNK) * CHUNK, rem)],
                )

        plsc.subcore_barrier()

        @pl.loop(0, nblk)
        def _blk(t):
            pltpu.sync_copy(src_hbm.at[wid, pl.ds(t * SBLK, SBLK)], src_v)
            pltpu.sync_copy(dst_hbm.at[wid, pl.ds(t * SBLK, SBLK)], dst_v)
            pltpu.sync_copy(val_hbm.at[wid, pl.ds(t * SBLK, SBLK)], val_v)

            @pl.loop(0, SBLK)
            def _chunk(j):
                pltpu.async_copy(support_hbm.at[src_v.at[j]], rows_v, sem).wait()

                @pl.loop(0, groups)
                def _grp(g):
                    vv = val_v[j, pl.ds(g * LANES, LANES)]
                    for k in range(LANES):
                        v16 = _bcast_lane(vv, k)
                        i = g * LANES + k
                        for c in range(dslices):
                            sl = pl.ds(c * LANES, LANES)
                            rows_v[i, sl] = rows_v[i, sl] * v16

                pltpu.sync_copy(rows_v, acc.at[dst_v.at[j]], add=True)

        plsc.subcore_barrier()

        @pl.when(sid < wtiles)
        def _writeback():
            pltpu.sync_copy(
                acc.at[pl.ds(sid * rows_per_tile, rows_per_tile)],
                out_hbm.at[cid, pl.ds(sid * rows_per_tile, rows_per_tile)],
            )

    return agg


def kernel(input_feature, edge_index, adj_values, W, b):
    n, d_in = input_feature.shape
    d_out = W.shape[1]
    e = adj_values.shape[0]

    support = _matmul(input_feature, W)

    eblk = NW * SBLK * CHUNK
    e_pad = -(-e // eblk) * eblk
    nch = e_pad // (NW * CHUNK)
    pad = e_pad - e
    src = edge_index[0]
    dst = edge_index[1]
    if pad:
        spread = (jnp.arange(pad, dtype=jnp.int32) * 97) % n
        src = jnp.concatenate([src, spread])
        dst = jnp.concatenate([dst, spread])
        adj_values = jnp.concatenate([adj_values, jnp.zeros((pad,), jnp.float32)])
    src3 = src.reshape(NW, nch, CHUNK)
    dst3 = dst.reshape(NW, nch, CHUNK)
    val3 = adj_values.reshape(NW, nch, CHUNK)

    partials = _make_aggregate(n, d_out, nch)(support, src3, dst3, val3)
    return _combine(partials, b.reshape(1, d_out))


# speedup vs baseline: 9.3280x; 1.4335x over previous
import functools

import jax
import jax.numpy as jnp
from jax import lax
from jax.experimental import pallas as pl
from jax.experimental.pallas import tpu as pltpu
from jax.experimental.pallas import tpu_sc as plsc

NC = 2
NS = 16
NW = NC * NS
CHUNK = 128
SBLK = 16
LANES = 16

_BCAST_DNUMS = lax.GatherDimensionNumbers(
    offset_dims=(), collapsed_slice_dims=(0,), start_index_map=(0,)
)


def _bcast_lane(vec16, k):
    idx = jnp.full((LANES, 1), k, jnp.int32)
    return lax.gather(
        vec16, idx, _BCAST_DNUMS, (1,),
        mode=lax.GatherScatterMode.PROMISE_IN_BOUNDS,
    )


def _matmul(x, w):
    n, d_in = x.shape
    d_out = w.shape[1]
    bm = 400
    grid = n // bm

    def body(x_ref, w_ref, o_ref):
        o_ref[:] = jnp.dot(x_ref[:], w_ref[:], preferred_element_type=jnp.float32)

    return pl.pallas_call(
        body,
        grid=(grid,),
        in_specs=[
            pl.BlockSpec((bm, d_in), lambda i: (i, 0)),
            pl.BlockSpec((d_in, d_out), lambda i: (0, 0)),
        ],
        out_specs=pl.BlockSpec((bm, d_out), lambda i: (i, 0)),
        out_shape=jax.ShapeDtypeStruct((n, d_out), jnp.float32),
    )(x, w)


def _combine(partials, b2d):
    _, n, d = partials.shape
    bm = 2000
    grid = n // bm

    def body(p_ref, b_ref, o_ref):
        o_ref[:] = p_ref[0] + p_ref[1] + b_ref[:]

    return pl.pallas_call(
        body,
        grid=(grid,),
        in_specs=[
            pl.BlockSpec((2, bm, d), lambda i: (0, i, 0)),
            pl.BlockSpec((1, d), lambda i: (0, 0)),
        ],
        out_specs=pl.BlockSpec((bm, d), lambda i: (i, 0)),
        out_shape=jax.ShapeDtypeStruct((n, d), jnp.float32),
    )(partials, b2d)


def _make_aggregate(n, d, nch):
    mesh = plsc.VectorSubcoreMesh(core_axis_name="c", subcore_axis_name="s")
    wtiles = 10
    rows_per_tile = n // wtiles
    dslices = d // LANES
    groups = CHUNK // LANES
    nblk = nch // SBLK

    @functools.partial(
        pl.kernel,
        out_type=jax.ShapeDtypeStruct((NC, n, d), jnp.float32),
        mesh=mesh,
        scratch_types=[
            pltpu.VMEM((SBLK, CHUNK), jnp.int32),
            pltpu.VMEM((SBLK, CHUNK), jnp.int32),
            pltpu.VMEM((SBLK, CHUNK), jnp.float32),
            pltpu.VMEM((2, CHUNK, d), jnp.float32),
            pltpu.VMEM_SHARED((n, d), jnp.float32),
            pltpu.SemaphoreType.DMA,
            pltpu.SemaphoreType.DMA,
        ],
    )
    def agg(support_hbm, src_hbm, dst_hbm, val_hbm, out_hbm,
            src_v, dst_v, val_v, rows2, acc, sem0, sem1):
        cid = lax.axis_index("c")
        sid = lax.axis_index("s")
        wid = sid * NC + cid

        sems = (sem0, sem1)

        @pl.loop(0, CHUNK)
        def _zero(r):
            for c in range(dslices):
                rows2[0, r, pl.ds(c * LANES, LANES)] = jnp.zeros((LANES,), jnp.float32)

        @pl.when(sid < wtiles)
        def _zacc_all():
            @pl.loop(0, rows_per_tile // CHUNK)
            def _zacc(r):
                pltpu.sync_copy(rows2.at[0],
                                acc.at[pl.ds(sid * rows_per_tile + r * CHUNK, CHUNK)])
            rem = rows_per_tile % CHUNK
            if rem:
                pltpu.sync_copy(
                    rows2.at[0, pl.ds(0, rem)],
                    acc.at[pl.ds(sid * rows_per_tile + (rows_per_tile // CHUNK) * CHUNK, rem)],
                )

        plsc.subcore_barrier()

        def _scale(r, j):
            @pl.loop(0, groups)
            def _grp(g):
                vv = val_v[j, pl.ds(g * LANES, LANES)]
                for k in range(LANES):
                    v16 = _bcast_lane(vv, k)
                    i = g * LANES + k
                    for c in range(dslices):
                        sl = pl.ds(c * LANES, LANES)
                        rows2[r, i, sl] = rows2[r, i, sl] * v16

        @pl.loop(0, nblk)
        def _blk(t):
            pltpu.sync_copy(src_hbm.at[wid, pl.ds(t * SBLK, SBLK)], src_v)
            pltpu.sync_copy(dst_hbm.at[wid, pl.ds(t * SBLK, SBLK)], dst_v)
            pltpu.sync_copy(val_hbm.at[wid, pl.ds(t * SBLK, SBLK)], val_v)

            pltpu.async_copy(support_hbm.at[src_v.at[0]], rows2.at[0], sem0)

            @pl.loop(0, SBLK // 2)
            def _step(s):
                for r in range(2):
                    j = s * 2 + r
                    pltpu.make_async_copy(
                        support_hbm.at[src_v.at[j]], rows2.at[r], sems[r]
                    ).wait()
                    if r == 0:
                        pltpu.async_copy(
                            support_hbm.at[src_v.at[j + 1]], rows2.at[1], sem1
                        )
                    else:
                        @pl.when(s < SBLK // 2 - 1)
                        def _prefetch():
                            pltpu.async_copy(
                                support_hbm.at[src_v.at[j + 1]], rows2.at[0], sem0
                            )
                    _scale(r, j)
                    pltpu.sync_copy(rows2.at[r], acc.at[dst_v.at[j]], add=True)

        plsc.subcore_barrier()

        @pl.when(sid < wtiles)
        def _writeback():
            pltpu.sync_copy(
                acc.at[pl.ds(sid * rows_per_tile, rows_per_tile)],
                out_hbm.at[cid, pl.ds(sid * rows_per_tile, rows_per_tile)],
            )

    return agg


def kernel(input_feature, edge_index, adj_values, W, b):
    n, d_in = input_feature.shape
    d_out = W.shape[1]
    e = adj_values.shape[0]

    support = _matmul(input_feature, W)

    eblk = NW * SBLK * CHUNK
    e_pad = -(-e // eblk) * eblk
    nch = e_pad // (NW * CHUNK)
    pad = e_pad - e
    src = edge_index[0]
    dst = edge_index[1]
    if pad:
        spread = (jnp.arange(pad, dtype=jnp.int32) * 97) % n
        src = jnp.concatenate([src, spread])
        dst = jnp.concatenate([dst, spread])
        adj_values = jnp.concatenate([adj_values, jnp.zeros((pad,), jnp.float32)])
    src3 = src.reshape(NW, nch, CHUNK)
    dst3 = dst.reshape(NW, nch, CHUNK)
    val3 = adj_values.reshape(NW, nch, CHUNK)

    partials = _make_aggregate(n, d_out, nch)(support, src3, dst3, val3)
    return _combine(partials, b.reshape(1, d_out))


# speedup vs baseline: 10.0608x; 1.0786x over previous
import functools

import jax
import jax.numpy as jnp
from jax import lax
from jax.experimental import pallas as pl
from jax.experimental.pallas import tpu as pltpu
from jax.experimental.pallas import tpu_sc as plsc

NC = 2
NS = 16
NW = NC * NS
CHUNK = 128
SBLK = 16
LANES = 16

_BCAST_DNUMS = lax.GatherDimensionNumbers(
    offset_dims=(), collapsed_slice_dims=(0,), start_index_map=(0,)
)


def _bcast_lane(vec16, k):
    idx = jnp.full((LANES, 1), k, jnp.int32)
    return lax.gather(
        vec16, idx, _BCAST_DNUMS, (1,),
        mode=lax.GatherScatterMode.PROMISE_IN_BOUNDS,
    )


def _matmul_combine(partials, w, b2d):
    _, n, d_in = partials.shape
    d_out = w.shape[1]
    bm = 2000
    grid = n // bm

    def body(p_ref, w_ref, b_ref, o_ref):
        agg = p_ref[0] + p_ref[1]
        o_ref[:] = (
            jnp.dot(agg, w_ref[:], preferred_element_type=jnp.float32) + b_ref[:]
        )

    return pl.pallas_call(
        body,
        grid=(grid,),
        in_specs=[
            pl.BlockSpec((2, bm, d_in), lambda i: (0, i, 0)),
            pl.BlockSpec((d_in, d_out), lambda i: (0, 0)),
            pl.BlockSpec((1, d_out), lambda i: (0, 0)),
        ],
        out_specs=pl.BlockSpec((bm, d_out), lambda i: (i, 0)),
        out_shape=jax.ShapeDtypeStruct((n, d_out), jnp.float32),
    )(partials, w, b2d)


def _make_aggregate(n, d, nch):
    mesh = plsc.VectorSubcoreMesh(core_axis_name="c", subcore_axis_name="s")
    wtiles = 10
    rows_per_tile = n // wtiles
    dslices = d // LANES
    groups = CHUNK // LANES
    nblk = nch // SBLK

    @functools.partial(
        pl.kernel,
        out_type=jax.ShapeDtypeStruct((NC, n, d), jnp.float32),
        mesh=mesh,
        scratch_types=[
            pltpu.VMEM((SBLK, CHUNK), jnp.int32),
            pltpu.VMEM((SBLK, CHUNK), jnp.int32),
            pltpu.VMEM((SBLK, CHUNK), jnp.float32),
            pltpu.VMEM((2, CHUNK, d), jnp.float32),
            pltpu.VMEM_SHARED((n, d), jnp.float32),
            pltpu.SemaphoreType.DMA,
            pltpu.SemaphoreType.DMA,
        ],
    )
    def agg(support_hbm, src_hbm, dst_hbm, val_hbm, out_hbm,
            src_v, dst_v, val_v, rows2, acc, sem0, sem1):
        cid = lax.axis_index("c")
        sid = lax.axis_index("s")
        wid = sid * NC + cid

        sems = (sem0, sem1)

        @pl.loop(0, CHUNK)
        def _zero(r):
            for c in range(dslices):
                rows2[0, r, pl.ds(c * LANES, LANES)] = jnp.zeros((LANES,), jnp.float32)

        @pl.when(sid < wtiles)
        def _zacc_all():
            @pl.loop(0, rows_per_tile // CHUNK)
            def _zacc(r):
                pltpu.sync_copy(rows2.at[0],
                                acc.at[pl.ds(sid * rows_per_tile + r * CHUNK, CHUNK)])
            rem = rows_per_tile % CHUNK
            if rem:
                pltpu.sync_copy(
                    rows2.at[0, pl.ds(0, rem)],
                    acc.at[pl.ds(sid * rows_per_tile + (rows_per_tile // CHUNK) * CHUNK, rem)],
                )

        plsc.subcore_barrier()

        def _scale(r, j):
            @pl.loop(0, groups)
            def _grp(g):
                vv = val_v[j, pl.ds(g * LANES, LANES)]
                for k in range(LANES):
                    v16 = _bcast_lane(vv, k)
                    i = g * LANES + k
                    for c in range(dslices):
                        sl = pl.ds(c * LANES, LANES)
                        rows2[r, i, sl] = rows2[r, i, sl] * v16

        @pl.loop(0, nblk)
        def _blk(t):
            pltpu.sync_copy(src_hbm.at[wid, pl.ds(t * SBLK, SBLK)], src_v)
            pltpu.sync_copy(dst_hbm.at[wid, pl.ds(t * SBLK, SBLK)], dst_v)
            pltpu.sync_copy(val_hbm.at[wid, pl.ds(t * SBLK, SBLK)], val_v)

            pltpu.async_copy(support_hbm.at[src_v.at[0]], rows2.at[0], sem0)

            @pl.loop(0, SBLK // 2)
            def _step(s):
                for r in range(2):
                    j = s * 2 + r
                    pltpu.make_async_copy(
                        support_hbm.at[src_v.at[j]], rows2.at[r], sems[r]
                    ).wait()
                    if r == 0:
                        pltpu.async_copy(
                            support_hbm.at[src_v.at[j + 1]], rows2.at[1], sem1
                        )
                    else:
                        @pl.when(s < SBLK // 2 - 1)
                        def _prefetch():
                            pltpu.async_copy(
                                support_hbm.at[src_v.at[j + 1]], rows2.at[0], sem0
                            )
                    _scale(r, j)
                    pltpu.sync_copy(rows2.at[r], acc.at[dst_v.at[j]], add=True)

        plsc.subcore_barrier()

        @pl.when(sid < wtiles)
        def _writeback():
            pltpu.sync_copy(
                acc.at[pl.ds(sid * rows_per_tile, rows_per_tile)],
                out_hbm.at[cid, pl.ds(sid * rows_per_tile, rows_per_tile)],
            )

    return agg


def kernel(input_feature, edge_index, adj_values, W, b):
    n, d_in = input_feature.shape
    d_out = W.shape[1]
    e = adj_values.shape[0]

    eblk = NW * SBLK * CHUNK
    e_pad = -(-e // eblk) * eblk
    nch = e_pad // (NW * CHUNK)
    pad = e_pad - e
    src = edge_index[0]
    dst = edge_index[1]
    if pad:
        spread = (jnp.arange(pad, dtype=jnp.int32) * 97) % n
        src = jnp.concatenate([src, spread])
        dst = jnp.concatenate([dst, spread])
        adj_values = jnp.concatenate([adj_values, jnp.zeros((pad,), jnp.float32)])
    src3 = src.reshape(NW, nch, CHUNK)
    dst3 = dst.reshape(NW, nch, CHUNK)
    val3 = adj_values.reshape(NW, nch, CHUNK)

    partials = _make_aggregate(n, d_in, nch)(input_feature, src3, dst3, val3)
    return _matmul_combine(partials, W, b.reshape(1, d_out))


# speedup vs baseline: 10.3677x; 1.0305x over previous
import functools

import jax
import jax.numpy as jnp
from jax import lax
from jax.experimental import pallas as pl
from jax.experimental.pallas import tpu as pltpu
from jax.experimental.pallas import tpu_sc as plsc

NC = 2
NS = 16
NW = NC * NS
CHUNK = 128
SBLK = 16
LANES = 16

_BCAST_DNUMS = lax.GatherDimensionNumbers(
    offset_dims=(), collapsed_slice_dims=(0,), start_index_map=(0,)
)


def _bcast_lane(vec16, k):
    idx = jnp.full((LANES, 1), k, jnp.int32)
    return lax.gather(
        vec16, idx, _BCAST_DNUMS, (1,),
        mode=lax.GatherScatterMode.PROMISE_IN_BOUNDS,
    )


def _matmul_combine(partials, w, b2d):
    _, n, d_in = partials.shape
    d_out = w.shape[1]
    bm = 2000
    grid = n // bm

    def body(p_ref, w_ref, b_ref, o_ref):
        agg = p_ref[0] + p_ref[1]
        o_ref[:] = (
            jnp.dot(agg, w_ref[:], preferred_element_type=jnp.float32) + b_ref[:]
        )

    return pl.pallas_call(
        body,
        grid=(grid,),
        in_specs=[
            pl.BlockSpec((2, bm, d_in), lambda i: (0, i, 0)),
            pl.BlockSpec((d_in, d_out), lambda i: (0, 0)),
            pl.BlockSpec((1, d_out), lambda i: (0, 0)),
        ],
        out_specs=pl.BlockSpec((bm, d_out), lambda i: (i, 0)),
        out_shape=jax.ShapeDtypeStruct((n, d_out), jnp.float32),
    )(partials, w, b2d)


def _make_aggregate(n, d, nch):
    mesh = plsc.VectorSubcoreMesh(core_axis_name="c", subcore_axis_name="s")
    wtiles = 10
    rows_per_tile = n // wtiles
    dslices = d // LANES
    groups = CHUNK // LANES
    nblk = nch // SBLK

    @functools.partial(
        pl.kernel,
        out_type=jax.ShapeDtypeStruct((NC, n, d), jnp.float32),
        mesh=mesh,
        scratch_types=[
            pltpu.VMEM((SBLK, CHUNK), jnp.int32),
            pltpu.VMEM((SBLK, CHUNK), jnp.int32),
            pltpu.VMEM((SBLK, CHUNK), jnp.float32),
            pltpu.VMEM((2, CHUNK, d), jnp.float32),
            pltpu.VMEM_SHARED((n, d), jnp.float32),
            pltpu.SemaphoreType.DMA,
            pltpu.SemaphoreType.DMA,
            pltpu.SemaphoreType.DMA,
            pltpu.SemaphoreType.DMA,
            pltpu.SemaphoreType.DMA,
        ],
    )
    def agg(support_hbm, src_hbm, dst_hbm, val_hbm, out_hbm,
            src_v, dst_v, val_v, rows2, acc, sem0, sem1, ssem0, ssem1, stsem):
        cid = lax.axis_index("c")
        sid = lax.axis_index("s")
        wid = sid * NC + cid

        sems = (sem0, sem1)

        @pl.loop(0, CHUNK)
        def _zero(r):
            for c in range(dslices):
                rows2[0, r, pl.ds(c * LANES, LANES)] = jnp.zeros((LANES,), jnp.float32)

        @pl.when(sid < wtiles)
        def _zacc_all():
            @pl.loop(0, rows_per_tile // CHUNK)
            def _zacc(r):
                pltpu.sync_copy(rows2.at[0],
                                acc.at[pl.ds(sid * rows_per_tile + r * CHUNK, CHUNK)])
            rem = rows_per_tile % CHUNK
            if rem:
                pltpu.sync_copy(
                    rows2.at[0, pl.ds(0, rem)],
                    acc.at[pl.ds(sid * rows_per_tile + (rows_per_tile // CHUNK) * CHUNK, rem)],
                )

        plsc.subcore_barrier()

        def _scale(r, j):
            @pl.loop(0, groups)
            def _grp(g):
                vv = val_v[j, pl.ds(g * LANES, LANES)]
                for k in range(LANES):
                    v16 = _bcast_lane(vv, k)
                    i = g * LANES + k
                    for c in range(dslices):
                        sl = pl.ds(c * LANES, LANES)
                        rows2[r, i, sl] = rows2[r, i, sl] * v16

        ssems = (ssem0, ssem1)

        @pl.loop(0, nblk)
        def _blk(t):
            pltpu.async_copy(src_hbm.at[wid, pl.ds(t * SBLK, SBLK)], src_v, stsem)
            pltpu.async_copy(dst_hbm.at[wid, pl.ds(t * SBLK, SBLK)], dst_v, stsem)
            pltpu.async_copy(val_hbm.at[wid, pl.ds(t * SBLK, SBLK)], val_v, stsem)
            pltpu.make_async_copy(src_hbm.at[wid, pl.ds(t * SBLK, SBLK)], src_v, stsem).wait()
            pltpu.make_async_copy(dst_hbm.at[wid, pl.ds(t * SBLK, SBLK)], dst_v, stsem).wait()
            pltpu.make_async_copy(val_hbm.at[wid, pl.ds(t * SBLK, SBLK)], val_v, stsem).wait()

            pltpu.async_copy(support_hbm.at[src_v.at[0]], rows2.at[0], sem0)

            @pl.loop(0, SBLK // 2)
            def _step(s):
                for r in range(2):
                    j = s * 2 + r
                    pltpu.make_async_copy(
                        support_hbm.at[src_v.at[j]], rows2.at[r], sems[r]
                    ).wait()
                    ro = 1 - r
                    if r == 0:
                        @pl.when(s > 0)
                        def _drain_prev():
                            pltpu.make_async_copy(
                                rows2.at[ro], acc.at[dst_v.at[j]], ssems[ro]
                            ).wait()
                        pltpu.async_copy(
                            support_hbm.at[src_v.at[j + 1]], rows2.at[1], sem1
                        )
                    else:
                        pltpu.make_async_copy(
                            rows2.at[ro], acc.at[dst_v.at[j]], ssems[ro]
                        ).wait()

                        @pl.when(s < SBLK // 2 - 1)
                        def _prefetch():
                            pltpu.async_copy(
                                support_hbm.at[src_v.at[j + 1]], rows2.at[0], sem0
                            )
                    _scale(r, j)
                    pltpu.async_copy(rows2.at[r], acc.at[dst_v.at[j]], ssems[r], add=True)

            pltpu.make_async_copy(rows2.at[1], acc.at[dst_v.at[0]], ssem1).wait()

        plsc.subcore_barrier()

        @pl.when(sid < wtiles)
        def _writeback():
            pltpu.sync_copy(
                acc.at[pl.ds(sid * rows_per_tile, rows_per_tile)],
                out_hbm.at[cid, pl.ds(sid * rows_per_tile, rows_per_tile)],
            )

    return agg


def kernel(input_feature, edge_index, adj_values, W, b):
    n, d_in = input_feature.shape
    d_out = W.shape[1]
    e = adj_values.shape[0]

    eblk = NW * SBLK * CHUNK
    e_pad = -(-e // eblk) * eblk
    nch = e_pad // (NW * CHUNK)
    pad = e_pad - e
    src = edge_index[0]
    dst = edge_index[1]
    if pad:
        spread = (jnp.arange(pad, dtype=jnp.int32) * 97) % n
        src = jnp.concatenate([src, spread])
        dst = jnp.concatenate([dst, spread])
        adj_values = jnp.concatenate([adj_values, jnp.zeros((pad,), jnp.float32)])
    src3 = src.reshape(NW, nch, CHUNK)
    dst3 = dst.reshape(NW, nch, CHUNK)
    val3 = adj_values.reshape(NW, nch, CHUNK)

    partials = _make_aggregate(n, d_in, nch)(input_feature, src3, dst3, val3)
    return _matmul_combine(partials, W, b.reshape(1, d_out))
